# Initial kernel scaffold; baseline (speedup 1.0000x reference)
import jax
import jax.numpy as jnp
from jax.experimental import pallas as pl


def kernel(indices, table):
    raise NotImplementedError("write your pallas kernel here")


# speedup vs baseline: 4.9079x; 4.9079x over previous
import functools

import jax
import jax.numpy as jnp
from jax import lax
from jax.experimental import pallas as pl
from jax.experimental.pallas import tpu as pltpu
from jax.experimental.pallas import tpu_sc as plsc

DIM = 32
NUM_CORES = 2
NUM_SUBCORES = 16
NW = NUM_CORES * NUM_SUBCORES

CHUNK = 1600


@functools.lru_cache(maxsize=None)
def _make_gather(n_rows: int):
    assert n_rows % NW == 0
    b_per_w = n_rows // NW
    assert b_per_w % CHUNK == 0
    n_chunks = b_per_w // CHUNK

    mesh = plsc.VectorSubcoreMesh(core_axis_name="c", subcore_axis_name="s")

    @functools.partial(
        pl.kernel,
        mesh=mesh,
        compiler_params=pltpu.CompilerParams(use_tc_tiling_on_sc=False),
        out_type=jax.ShapeDtypeStruct((n_rows, DIM), jnp.float32),
        scratch_types=[
            pltpu.VMEM((CHUNK,), jnp.int32),
            pltpu.VMEM((CHUNK, DIM), jnp.float32),
            pltpu.SemaphoreType.DMA,
        ],
    )
    def gather(idx_hbm, table_hbm, out_hbm, idx_v, rows_v, sem):
        wid = lax.axis_index("s") * NUM_CORES + lax.axis_index("c")
        base = wid * b_per_w

        def body(i, carry):
            off = base + i * CHUNK
            pltpu.sync_copy(idx_hbm.at[pl.ds(off, CHUNK)], idx_v)
            pltpu.async_copy(table_hbm.at[idx_v], rows_v, sem).wait()
            pltpu.sync_copy(rows_v, out_hbm.at[pl.ds(off, CHUNK)])
            return carry

        lax.fori_loop(0, n_chunks, body, 0)

    return gather


def kernel(indices, table):
    batch, hist = indices.shape
    flat_idx = indices.reshape(-1).astype(jnp.int32)
    out = _make_gather(batch * hist)(flat_idx, table)
    return out.reshape(batch, hist, DIM)


# speedup vs baseline: 5.0428x; 1.0275x over previous
import functools

import jax
import jax.numpy as jnp
from jax import lax
from jax.experimental import pallas as pl
from jax.experimental.pallas import tpu as pltpu
from jax.experimental.pallas import tpu_sc as plsc

DIM = 32
NUM_CORES = 2
NUM_SUBCORES = 16
NW = NUM_CORES * NUM_SUBCORES

CHUNK = 1600
NBUF = 2


@functools.lru_cache(maxsize=None)
def _make_gather(n_rows: int):
    assert n_rows % NW == 0
    b_per_w = n_rows // NW
    assert b_per_w % (CHUNK * NBUF) == 0
    n_chunks = b_per_w // CHUNK

    mesh = plsc.VectorSubcoreMesh(core_axis_name="c", subcore_axis_name="s")

    scratch = (
        [pltpu.VMEM((CHUNK,), jnp.int32) for _ in range(NBUF)]
        + [pltpu.VMEM((CHUNK, DIM), jnp.float32) for _ in range(NBUF)]
        + [pltpu.SemaphoreType.DMA for _ in range(3 * NBUF)]
    )

    @functools.partial(
        pl.kernel,
        mesh=mesh,
        compiler_params=pltpu.CompilerParams(use_tc_tiling_on_sc=False),
        out_type=jax.ShapeDtypeStruct((n_rows, DIM), jnp.float32),
        scratch_types=scratch,
    )
    def gather(idx_hbm, table_hbm, out_hbm, *bufs):
        idx_v = bufs[:NBUF]
        rows_v = bufs[NBUF:2 * NBUF]
        isem = bufs[2 * NBUF:3 * NBUF]
        gsem = bufs[3 * NBUF:4 * NBUF]
        wsem = bufs[4 * NBUF:5 * NBUF]

        wid = lax.axis_index("s") * NUM_CORES + lax.axis_index("c")
        base = wid * b_per_w

        def idx_copy(g, b):
            return pltpu.make_async_copy(
                idx_hbm.at[pl.ds(base + g * CHUNK, CHUNK)], idx_v[b], isem[b])

        def gather_copy(b):
            return pltpu.make_async_copy(table_hbm.at[idx_v[b]], rows_v[b], gsem[b])

        def write_copy(g, b):
            return pltpu.make_async_copy(
                rows_v[b], out_hbm.at[pl.ds(base + g * CHUNK, CHUNK)], wsem[b])

        for b in range(NBUF):
            idx_copy(b, b).start()
        for b in range(NBUF):
            idx_copy(b, b).wait()
            gather_copy(b).start()

        def body(t, carry):
            g0 = t * NBUF
            for b in range(NBUF):
                g = g0 + b
                gather_copy(b).wait()
                write_copy(g, b).start()

                @pl.when(g + NBUF < n_chunks)
                def _():
                    idx_copy(g + NBUF, b).start()

            for b in range(NBUF):
                g = g0 + b

                @pl.when(g + NBUF < n_chunks)
                def _():
                    idx_copy(g + NBUF, b).wait()
                    write_copy(g, b).wait()
                    gather_copy(b).start()

            return carry

        lax.fori_loop(0, n_chunks // NBUF, body, 0)

        for b in range(NBUF):
            write_copy(n_chunks - NBUF + b, b).wait()

    return gather


def kernel(indices, table):
    batch, hist = indices.shape
    flat_idx = indices.reshape(-1).astype(jnp.int32)
    out = _make_gather(batch * hist)(flat_idx, table)
    return out.reshape(batch, hist, DIM)
